# baseline (device time: 10549 ns/iter reference)
import jax
import jax.numpy as jnp
from jax import lax
from jax.experimental import pallas as pl
from jax.experimental.pallas import tpu as pltpu

N_DEV = 8
N_TOK = 256
D_IN = 128
D_OUT = 256
N_EXP = 16
EXPERTS_PER_DEV = 2
CAP = 12
SLOTS = EXPERTS_PER_DEV * CAP
SLOTS_PAD = 32
GSLOTS = N_DEV * SLOTS_PAD


def kernel(x, router_W, route_idx, expert_W):
    del router_W

    idx_row = route_idx.reshape(1, N_TOK)
    c = jnp.arange(GSLOTS, dtype=jnp.int32)
    s = c % SLOTS_PAD
    e_of = (c // SLOTS_PAD) * EXPERTS_PER_DEV + s // CAP
    valid = s < SLOTS
    ET = (
        jnp.arange(N_EXP, dtype=jnp.int32)[:, None]
        == jnp.where(valid, e_of, -1)[None, :]
    ).astype(jnp.float32)
    cmod = jnp.where(valid, s % CAP, -2).astype(jnp.float32).reshape(1, GSLOTS)
    dev_c = (c // SLOTS_PAD).astype(jnp.int32).reshape(GSLOTS, 1)

    def body(x_ref, idxc_ref, idxr_ref, cmod_ref, et_ref, devc_ref, w_ref,
             out_ref, allcomp, send_sems, recv_sems):
        my_i = lax.axis_index("i")

        barrier_sem = pltpu.get_barrier_semaphore()
        for k in range(1, N_DEV):
            pl.semaphore_signal(
                barrier_sem, inc=1,
                device_id=(my_i ^ k,), device_id_type=pl.DeviceIdType.MESH,
            )

        idx_r = idxr_ref[:, :]
        jr = lax.broadcasted_iota(jnp.int32, (EXPERTS_PER_DEV, N_TOK), 0)
        matchT = idx_r == my_i * EXPERTS_PER_DEV + jr
        matchT_f = jnp.where(matchT, 1.0, 0.0).astype(jnp.float32)
        r_i = lax.broadcasted_iota(jnp.int32, (N_TOK, N_TOK), 0)
        c_i = lax.broadcasted_iota(jnp.int32, (N_TOK, N_TOK), 1)
        U = jnp.where(r_i <= c_i, 1.0, 0.0).astype(jnp.float32)
        rankT = jnp.dot(matchT_f, U, preferred_element_type=jnp.float32)
        s32 = lax.broadcasted_iota(jnp.int32, (SLOTS_PAD, EXPERTS_PER_DEV), 0)
        j32 = lax.broadcasted_iota(jnp.int32, (SLOTS_PAD, EXPERTS_PER_DEV), 1)
        E32 = jnp.where(
            (s32 >= CAP * j32) & (s32 < CAP * j32 + CAP), 1.0, 0.0
        ).astype(jnp.float32)
        rankT32 = jnp.dot(E32, rankT, preferred_element_type=jnp.float32)
        matchT32 = jnp.dot(E32, matchT_f, preferred_element_type=jnp.float32)
        srow = lax.broadcasted_iota(jnp.int32, (SLOTS_PAD, N_TOK), 0)
        rmod = jnp.where(srow < CAP, srow, srow - CAP).astype(jnp.float32)
        C = jnp.where(
            (matchT32 > 0.5) & (rankT32 == rmod + 1.0), 1.0, 0.0
        ).astype(jnp.float32)

        xc = jnp.dot(C, x_ref[:, :], preferred_element_type=jnp.float32)
        comp = jnp.concatenate(
            [
                jnp.dot(xc[:CAP, :], w_ref[0, :, :],
                        preferred_element_type=jnp.float32),
                jnp.dot(xc[CAP:, :], w_ref[1, :, :],
                        preferred_element_type=jnp.float32),
            ],
            axis=0,
        )
        allcomp[my_i, :, :] = comp.astype(jnp.bfloat16)

        pl.semaphore_wait(barrier_sem, N_DEV - 1)
        sends = []
        for k in (6, 2, 5, 7, 1, 3, 4):
            s = pltpu.make_async_remote_copy(
                src_ref=allcomp.at[my_i],
                dst_ref=allcomp.at[my_i],
                send_sem=send_sems.at[k],
                recv_sem=recv_sems.at[k],
                device_id=(my_i ^ k,),
                device_id_type=pl.DeviceIdType.MESH,
            )
            s.start()
            sends.append(s)

        idx_c = idxc_ref[:, :]
        e16 = lax.broadcasted_iota(jnp.int32, (N_TOK, N_EXP), 1)
        match_all = idx_c == e16
        match_all_f = jnp.where(match_all, 1.0, 0.0).astype(jnp.float32)
        tril = jnp.where(c_i <= r_i, 1.0, 0.0).astype(jnp.float32)
        rank_all = jnp.dot(tril, match_all_f,
                           preferred_element_type=jnp.float32)
        rank_exp = jnp.dot(rank_all, et_ref[:, :],
                           preferred_element_type=jnp.float32)
        match_exp = jnp.dot(match_all_f, et_ref[:, :],
                            preferred_element_type=jnp.float32)
        P = jnp.where(
            (match_exp > 0.5) & (rank_exp == cmod_ref[:, :] + 1.0), 1.0, 0.0
        ).astype(jnp.bfloat16)

        def wait_recv(k):
            pltpu.make_async_remote_copy(
                src_ref=allcomp.at[my_i],
                dst_ref=allcomp.at[my_i ^ k],
                send_sem=send_sems.at[k],
                recv_sem=recv_sems.at[k],
                device_id=(my_i ^ k,),
                device_id_type=pl.DeviceIdType.MESH,
            ).wait_recv()

        nearmask = (
            (devc_ref[:, :] == my_i)
            | (devc_ref[:, :] == my_i ^ 1)
            | (devc_ref[:, :] == my_i ^ 3)
            | (devc_ref[:, :] == my_i ^ 4)
        )

        for k in (1, 3, 4):
            wait_recv(k)
        allflat = allcomp[:, :, :].reshape(GSLOTS, D_OUT)
        near = jnp.where(nearmask, allflat, jnp.bfloat16(0.0))
        out_near = jnp.dot(P, near, preferred_element_type=jnp.float32)

        for k in (2, 5, 7, 6):
            wait_recv(k)
        allflat2 = allcomp[:, :, :].reshape(GSLOTS, D_OUT)
        far = jnp.where(nearmask, jnp.bfloat16(0.0), allflat2)
        out_ref[:, :] = out_near + jnp.dot(
            P, far, preferred_element_type=jnp.float32
        )

        for s in sends:
            s.wait_send()

    return pl.pallas_call(
        body,
        out_shape=jax.ShapeDtypeStruct((N_TOK, D_OUT), jnp.float32),
        in_specs=[pl.BlockSpec(memory_space=pltpu.VMEM)] * 7,
        out_specs=pl.BlockSpec(memory_space=pltpu.VMEM),
        scratch_shapes=[
            pltpu.VMEM((N_DEV, SLOTS_PAD, D_OUT), jnp.bfloat16),
            pltpu.SemaphoreType.DMA((N_DEV,)),
            pltpu.SemaphoreType.DMA((N_DEV,)),
        ],
        compiler_params=pltpu.CompilerParams(collective_id=0),
    )(x, route_idx, idx_row, cmod, ET, dev_c, expert_W)


# device time: 9479 ns/iter; 1.1129x vs baseline; 1.1129x over previous
import jax
import jax.numpy as jnp
from jax import lax
from jax.experimental import pallas as pl
from jax.experimental.pallas import tpu as pltpu

N_DEV = 8
N_TOK = 256
D_IN = 128
D_OUT = 256
N_EXP = 16
EXPERTS_PER_DEV = 2
CAP = 12
SLOTS = EXPERTS_PER_DEV * CAP
SLOTS_PAD = 32
GSLOTS = N_DEV * SLOTS_PAD


def kernel(x, router_W, route_idx, expert_W):
    del router_W

    idx_row = route_idx.reshape(1, N_TOK)
    c = jnp.arange(GSLOTS, dtype=jnp.int32)
    s = c % SLOTS_PAD
    e_of = (c // SLOTS_PAD) * EXPERTS_PER_DEV + s // CAP
    valid = s < SLOTS
    ET = (
        jnp.arange(N_EXP, dtype=jnp.int32)[:, None]
        == jnp.where(valid, e_of, -1)[None, :]
    ).astype(jnp.float32)
    cmod = jnp.where(valid, s % CAP, -2).astype(jnp.float32).reshape(1, GSLOTS)

    def body(x_ref, idxc_ref, idxr_ref, cmod_ref, et_ref, w_ref,
             out_ref, allcomp, send_sems, recv_sems):
        my_i = lax.axis_index("i")

        barrier_sem = pltpu.get_barrier_semaphore()
        for k in range(1, N_DEV):
            pl.semaphore_signal(
                barrier_sem, inc=1,
                device_id=(my_i ^ k,), device_id_type=pl.DeviceIdType.MESH,
            )

        idx_r = idxr_ref[:, :]
        jr = lax.broadcasted_iota(jnp.int32, (EXPERTS_PER_DEV, N_TOK), 0)
        matchT = idx_r == my_i * EXPERTS_PER_DEV + jr
        matchT_f = jnp.where(matchT, 1.0, 0.0).astype(jnp.float32)
        r_i = lax.broadcasted_iota(jnp.int32, (N_TOK, N_TOK), 0)
        c_i = lax.broadcasted_iota(jnp.int32, (N_TOK, N_TOK), 1)
        U = jnp.where(r_i <= c_i, 1.0, 0.0).astype(jnp.float32)
        rankT = jnp.dot(matchT_f, U, preferred_element_type=jnp.float32)
        s32 = lax.broadcasted_iota(jnp.int32, (SLOTS_PAD, EXPERTS_PER_DEV), 0)
        j32 = lax.broadcasted_iota(jnp.int32, (SLOTS_PAD, EXPERTS_PER_DEV), 1)
        E32 = jnp.where(
            (s32 >= CAP * j32) & (s32 < CAP * j32 + CAP), 1.0, 0.0
        ).astype(jnp.float32)
        rankT32 = jnp.dot(E32, rankT, preferred_element_type=jnp.float32)
        matchT32 = jnp.dot(E32, matchT_f, preferred_element_type=jnp.float32)
        srow = lax.broadcasted_iota(jnp.int32, (SLOTS_PAD, N_TOK), 0)
        rmod = jnp.where(srow < CAP, srow, srow - CAP).astype(jnp.float32)
        C = jnp.where(
            (matchT32 > 0.5) & (rankT32 == rmod + 1.0), 1.0, 0.0
        ).astype(jnp.float32)

        xc = jnp.dot(C, x_ref[:, :], preferred_element_type=jnp.float32)
        allcomp[my_i, :CAP, :] = jnp.dot(
            xc[:CAP, :], w_ref[0, :, :], preferred_element_type=jnp.float32
        ).astype(jnp.bfloat16)
        allcomp[my_i, CAP:, :] = jnp.dot(
            xc[CAP:, :], w_ref[1, :, :], preferred_element_type=jnp.float32
        ).astype(jnp.bfloat16)

        pl.semaphore_wait(barrier_sem, N_DEV - 1)
        sends = []
        for k in (6, 2, 5, 7, 1, 3, 4):
            s = pltpu.make_async_remote_copy(
                src_ref=allcomp.at[my_i],
                dst_ref=allcomp.at[my_i],
                send_sem=send_sems.at[k],
                recv_sem=recv_sems.at[k],
                device_id=(my_i ^ k,),
                device_id_type=pl.DeviceIdType.MESH,
            )
            s.start()
            sends.append(s)

        idx_c = idxc_ref[:, :]
        e16 = lax.broadcasted_iota(jnp.int32, (N_TOK, N_EXP), 1)
        match_all = idx_c == e16
        match_all_f = jnp.where(match_all, 1.0, 0.0).astype(jnp.float32)
        tril = jnp.where(c_i <= r_i, 1.0, 0.0).astype(jnp.float32)
        rank_all = jnp.dot(tril, match_all_f,
                           preferred_element_type=jnp.float32)
        rank_exp = jnp.dot(rank_all, et_ref[:, :],
                           preferred_element_type=jnp.float32)
        match_exp = jnp.dot(match_all_f, et_ref[:, :],
                            preferred_element_type=jnp.float32)
        P = jnp.where(
            (match_exp > 0.5) & (rank_exp == cmod_ref[:, :] + 1.0), 1.0, 0.0
        ).astype(jnp.bfloat16)

        for k in range(1, N_DEV):
            pltpu.make_async_remote_copy(
                src_ref=allcomp.at[my_i],
                dst_ref=allcomp.at[my_i ^ k],
                send_sem=send_sems.at[k],
                recv_sem=recv_sems.at[k],
                device_id=(my_i ^ k,),
                device_id_type=pl.DeviceIdType.MESH,
            ).wait_recv()

        allflat = allcomp[:, :, :].reshape(GSLOTS, D_OUT)
        out_ref[:, :] = jnp.dot(P, allflat,
                                preferred_element_type=jnp.float32)

        for s in sends:
            s.wait_send()

    return pl.pallas_call(
        body,
        out_shape=jax.ShapeDtypeStruct((N_TOK, D_OUT), jnp.float32),
        in_specs=[pl.BlockSpec(memory_space=pltpu.VMEM)] * 6,
        out_specs=pl.BlockSpec(memory_space=pltpu.VMEM),
        scratch_shapes=[
            pltpu.VMEM((N_DEV, SLOTS_PAD, D_OUT), jnp.bfloat16),
            pltpu.SemaphoreType.DMA((N_DEV,)),
            pltpu.SemaphoreType.DMA((N_DEV,)),
        ],
        compiler_params=pltpu.CompilerParams(collective_id=0),
    )(x, route_idx, idx_row, cmod, ET, expert_W)
